# baseline (speedup 1.0000x reference)
import jax
import jax.numpy as jnp
from jax.experimental import pallas as pl
from jax.experimental.pallas import tpu as pltpu

DEPTH = 4
NUM_CODES = 1024
DIM = 64
NSUB = 6
SUB_T = 384
BLOCK_T = NSUB * SUB_T
N_TOKENS = 9216


def _rvq_kernel(x_ref, cb_ref, quant_ref, idx_ref, loss_ref,
                cb2_ref, cbn_ref):
    @pl.when(pl.program_id(0) == 0)
    def _init():
        loss_ref[...] = jnp.zeros((1, 1), dtype=jnp.float32)
        cb = cb_ref[...]
        cb_hi = cb.astype(jnp.bfloat16)
        rem1 = cb - cb_hi.astype(jnp.float32)
        cb_mid = rem1.astype(jnp.bfloat16)
        cb_lo = (rem1 - cb_mid.astype(jnp.float32)).astype(jnp.bfloat16)
        cb2_ref[...] = jnp.concatenate([cb_hi, cb_mid, cb_lo], axis=2)
        ones_row = jnp.ones((1, DIM), dtype=jnp.float32)
        for g in range(DEPTH):
            cbg = cb[g]
            cbn_ref[g, 0:1, :] = jax.lax.dot_general(
                ones_row, cbg * cbg, (((1,), (1,)), ((), ())),
                precision=jax.lax.Precision.HIGHEST,
                preferred_element_type=jnp.float32)

    def half_stage(g, r, cbn):
        dots = jax.lax.dot_general(
            -2.0 * r, cb_ref[g], (((1,), (1,)), ((), ())),
            preferred_element_type=jnp.float32)
        scores = dots + cbn
        idx = jnp.argmin(scores, axis=1, keepdims=True)
        iota = jax.lax.broadcasted_iota(jnp.int32, (SUB_T, NUM_CODES), 1)
        onehot = (iota == idx).astype(jnp.float32).astype(jnp.bfloat16)
        q2 = jax.lax.dot_general(
            onehot, cb2_ref[g], (((1,), (0,)), ((), ())),
            preferred_element_type=jnp.float32)
        q = q2[:, :DIM] + q2[:, DIM:2 * DIM] + q2[:, 2 * DIM:]
        return q, idx

    def stage(g, carry):
        rs, qts, loss, idxs = carry
        cbn = cbn_ref[g, 0:1, :]
        new_rs, new_qts, new_idxs = [], [], []
        for s in range(NSUB):
            q, idx = half_stage(g, rs[s], cbn)
            d = rs[s] - q
            loss = loss + jnp.sum(d * d)
            new_rs.append(d)
            new_qts.append(qts[s] + q)
            new_idxs.append([jnp.where(g == k, idx, idxs[s][k])
                             for k in range(DEPTH)])
        return new_rs, new_qts, loss, new_idxs

    xs = [x_ref[pl.ds(s * SUB_T, SUB_T), :] for s in range(NSUB)]
    z = [[jnp.zeros((SUB_T, 1), dtype=jnp.int32)] * DEPTH
         for _ in range(NSUB)]
    _, qts, loss, idxs = jax.lax.fori_loop(
        0, DEPTH, stage,
        (xs, [jnp.zeros_like(x) for x in xs],
         jnp.zeros((), jnp.float32), z))
    for s in range(NSUB):
        quant_ref[pl.ds(s * SUB_T, SUB_T), :] = qts[s]
        idx_ref[pl.ds(s * SUB_T, SUB_T), :] = jnp.concatenate(
            idxs[s], axis=1)
    loss_ref[...] += loss.reshape(1, 1)

    @pl.when(pl.program_id(0) == pl.num_programs(0) - 1)
    def _finish():
        loss_ref[...] *= 1.25 / (N_TOKENS * DIM)


def kernel(embeds, codebook):
    B, T, D = embeds.shape
    N = B * T
    x = embeds.reshape(N, D)
    grid = (N // BLOCK_T,)
    quant, idx, loss_acc = pl.pallas_call(
        _rvq_kernel,
        grid=grid,
        in_specs=[
            pl.BlockSpec((BLOCK_T, D), lambda i: (i, 0)),
            pl.BlockSpec((DEPTH, NUM_CODES, D), lambda i: (0, 0, 0)),
        ],
        out_specs=[
            pl.BlockSpec((BLOCK_T, D), lambda i: (i, 0)),
            pl.BlockSpec((BLOCK_T, DEPTH), lambda i: (i, 0)),
            pl.BlockSpec((1, 1), lambda i: (0, 0)),
        ],
        out_shape=[
            jax.ShapeDtypeStruct((N, D), jnp.float32),
            jax.ShapeDtypeStruct((N, DEPTH), jnp.int32),
            jax.ShapeDtypeStruct((1, 1), jnp.float32),
        ],
        scratch_shapes=[
            pltpu.VMEM((DEPTH, NUM_CODES, 3 * DIM), jnp.bfloat16),
            pltpu.VMEM((DEPTH, 8, NUM_CODES), jnp.float32),
        ],
    )(x, codebook)
    quantized = quant.reshape(B, T, D)
    indices = idx.reshape(B, T, DEPTH)
    loss = loss_acc.reshape(())
    return quantized, indices, loss


# speedup vs baseline: 1.0363x; 1.0363x over previous
import jax
import jax.numpy as jnp
from jax.experimental import pallas as pl
from jax.experimental.pallas import tpu as pltpu

DEPTH = 4
NUM_CODES = 1024
DIM = 64
NSUB = 4
SUB_T = 384
BLOCK_T = NSUB * SUB_T
N_TOKENS = 9216


def _rvq_kernel(x_ref, cb_ref, quant_ref, idx_ref, loss_ref,
                cb2_ref, cbn_ref):
    @pl.when(pl.program_id(0) == 0)
    def _init():
        loss_ref[...] = jnp.zeros((1, 1), dtype=jnp.float32)
        cb = cb_ref[...]
        cb_hi = cb.astype(jnp.bfloat16)
        rem1 = cb - cb_hi.astype(jnp.float32)
        cb_mid = rem1.astype(jnp.bfloat16)
        cb_lo = (rem1 - cb_mid.astype(jnp.float32)).astype(jnp.bfloat16)
        cb2_ref[...] = jnp.concatenate([cb_hi, cb_mid, cb_lo], axis=2)
        ones_row = jnp.ones((1, DIM), dtype=jnp.float32)
        for g in range(DEPTH):
            cbg = cb[g]
            cbn_ref[g, 0:1, :] = jax.lax.dot_general(
                ones_row, cbg * cbg, (((1,), (1,)), ((), ())),
                precision=jax.lax.Precision.HIGHEST,
                preferred_element_type=jnp.float32)

    def half_stage(g, r, cbn):
        dots = jax.lax.dot_general(
            -2.0 * r, cb_ref[g], (((1,), (1,)), ((), ())),
            preferred_element_type=jnp.float32)
        scores = dots + cbn
        idx = jnp.argmin(scores, axis=1, keepdims=True)
        iota = jax.lax.broadcasted_iota(jnp.int32, (SUB_T, NUM_CODES), 1)
        onehot = (iota == idx).astype(jnp.float32).astype(jnp.bfloat16)
        q2 = jax.lax.dot_general(
            onehot, cb2_ref[g], (((1,), (0,)), ((), ())),
            preferred_element_type=jnp.float32)
        q = q2[:, :DIM] + q2[:, DIM:2 * DIM] + q2[:, 2 * DIM:]
        return q, idx

    def stage(g, carry):
        rs, qts, loss, idxs = carry
        cbn = cbn_ref[g, 0:1, :]
        new_rs, new_qts, new_idxs = [], [], []
        for s in range(NSUB):
            q, idx = half_stage(g, rs[s], cbn)
            d = rs[s] - q
            loss = loss + jnp.sum(d * d)
            new_rs.append(d)
            new_qts.append(qts[s] + q)
            new_idxs.append([jnp.where(g == k, idx, idxs[s][k])
                             for k in range(DEPTH)])
        return new_rs, new_qts, loss, new_idxs

    xs = [x_ref[pl.ds(s * SUB_T, SUB_T), :] for s in range(NSUB)]
    z = [[jnp.zeros((SUB_T, 1), dtype=jnp.int32)] * DEPTH
         for _ in range(NSUB)]
    _, qts, loss, idxs = jax.lax.fori_loop(
        0, DEPTH, stage,
        (xs, [jnp.zeros_like(x) for x in xs],
         jnp.zeros((), jnp.float32), z))
    for s in range(NSUB):
        quant_ref[pl.ds(s * SUB_T, SUB_T), :] = qts[s]
        idx_ref[pl.ds(s * SUB_T, SUB_T), :] = jnp.concatenate(
            idxs[s], axis=1)
    loss_ref[...] += loss.reshape(1, 1)

    @pl.when(pl.program_id(0) == pl.num_programs(0) - 1)
    def _finish():
        loss_ref[...] *= 1.25 / (N_TOKENS * DIM)


def kernel(embeds, codebook):
    B, T, D = embeds.shape
    N = B * T
    x = embeds.reshape(N, D)
    grid = (N // BLOCK_T,)
    quant, idx, loss_acc = pl.pallas_call(
        _rvq_kernel,
        grid=grid,
        in_specs=[
            pl.BlockSpec((BLOCK_T, D), lambda i: (i, 0)),
            pl.BlockSpec((DEPTH, NUM_CODES, D), lambda i: (0, 0, 0)),
        ],
        out_specs=[
            pl.BlockSpec((BLOCK_T, D), lambda i: (i, 0)),
            pl.BlockSpec((BLOCK_T, DEPTH), lambda i: (i, 0)),
            pl.BlockSpec((1, 1), lambda i: (0, 0)),
        ],
        out_shape=[
            jax.ShapeDtypeStruct((N, D), jnp.float32),
            jax.ShapeDtypeStruct((N, DEPTH), jnp.int32),
            jax.ShapeDtypeStruct((1, 1), jnp.float32),
        ],
        scratch_shapes=[
            pltpu.VMEM((DEPTH, NUM_CODES, 3 * DIM), jnp.bfloat16),
            pltpu.VMEM((DEPTH, 8, NUM_CODES), jnp.float32),
        ],
    )(x, codebook)
    quantized = quant.reshape(B, T, D)
    indices = idx.reshape(B, T, DEPTH)
    loss = loss_acc.reshape(())
    return quantized, indices, loss
